# baseline (device time: 103006 ns/iter reference)
import jax
import jax.numpy as jnp
from jax import lax
from jax.experimental import pallas as pl
from jax.experimental.pallas import tpu as pltpu

N_DEV = 8
SQ = 2048
SKV = 2048
D_MODEL = 1024
H_LOC = 8
DH = 128
SCALE = 0.08838834764831843
BAND = 128
NGLOB = 32
GLOB = 128

_PARTS = []
_r = 0
for _n in (64, 320, 384, 256, 256, 256, 256, 256):
    _PARTS.append((_r, _n))
    _r += _n
assert _r == SQ
N_PARTS = len(_PARTS)
_A2A_PARTS = frozenset({6, 7})

_ORDERS = (("x", "y", "z"), ("y", "z", "x"), ("z", "x", "y"))
_XORS = {"x": 1, "y": 3, "z": 4}

_ROFF = {}
_acc = 0
for _p, (_r0, _n) in enumerate(_PARTS):
    if _p in _A2A_PARTS:
        _ROFF[(_p, 0)] = _acc
        _acc += _n
    else:
        for _s in range(3):
            _ROFF[(_p, _s)] = _acc
            _acc += _n >> (_s + 1)
_RBUF_ROWS = _acc

_N_SEMS = 36 + 2 * 14


def _softmax_ctx(s, mask, w_slices, v_slices):
    s = jnp.where(mask, s, -1e9)
    w = jnp.exp(s - jnp.max(s, axis=1, keepdims=True))
    w = (w / jnp.sum(w, axis=1, keepdims=True)).astype(jnp.bfloat16)
    ctx = None
    for (lo, hi), v in zip(w_slices, v_slices):
        part = jnp.dot(w[:, lo:hi], v, preferred_element_type=jnp.float32)
        ctx = part if ctx is None else ctx + part
    return ctx.astype(jnp.bfloat16)


def _dot_t(a, bmat):
    return lax.dot_general(
        a, bmat, (((1,), (1,)), ((), ())), preferred_element_type=jnp.float32)


def _compute_part(p, x_ref, wq_ref, k_ref, v_ref, wo_ref, c_ref):
    r0, n = _PARTS[p]
    q = jnp.dot(x_ref[r0:r0 + n, :].astype(jnp.bfloat16), wq_ref[...],
                preferred_element_type=jnp.float32).astype(jnp.bfloat16)
    if p == 0:
        qi = lax.broadcasted_iota(jnp.int32, (n, SKV), 0)
        ki = lax.broadcasted_iota(jnp.int32, (n, SKV), 1)
        mask = (jnp.abs(qi - ki) <= BAND) | (ki < NGLOB) | (qi < NGLOB)
        ctx_parts = []
        for h in range(H_LOC):
            s = _dot_t(q[:, h * DH:(h + 1) * DH], k_ref[h]) * SCALE
            ctx_parts.append(_softmax_ctx(s, mask, [(0, SKV)], [v_ref[h]]))
    else:
        win = n + 2 * BAND
        ws = max(0, min(r0 - BAND, SKV - win))
        use_glob = ws > 0
        width = win + (GLOB if use_glob else 0)
        qi = lax.broadcasted_iota(jnp.int32, (n, width), 0) + r0
        ki = lax.broadcasted_iota(jnp.int32, (n, width), 1)
        if use_glob:
            ki = jnp.where(ki < win, ki + ws, ki - win)
        else:
            ki = ki + ws
        mask = (jnp.abs(qi - ki) <= BAND) | (ki < NGLOB)
        ctx_parts = []
        for h in range(H_LOC):
            qh = q[:, h * DH:(h + 1) * DH]
            pieces = [_dot_t(qh, k_ref[h, ws:ws + win, :])]
            w_slices = [(0, win)]
            v_slices = [v_ref[h, ws:ws + win, :]]
            if use_glob:
                pieces.append(_dot_t(qh, k_ref[h, :GLOB, :]))
                w_slices.append((win, win + GLOB))
                v_slices.append(v_ref[h, :GLOB, :])
            s = jnp.concatenate(pieces, axis=1) * SCALE if use_glob \
                else pieces[0] * SCALE
            ctx_parts.append(_softmax_ctx(s, mask, w_slices, v_slices))
    c_ref[r0:r0 + n, :] = jnp.dot(
        jnp.concatenate(ctx_parts, axis=1), wo_ref[...],
        preferred_element_type=jnp.float32).astype(jnp.bfloat16)


def _fused_body(x_ref, wq_ref, k_ref, v_ref, wo_ref, c_ref,
                rbuf, send_sems, recv_sems):
    pos = lax.axis_index("i")
    cy = (pos >> 1) & 1
    cx = (pos & 1) ^ cy
    cz = (pos >> 2) & 1
    bits = {"x": cx, "y": cy, "z": cz}

    barrier = pltpu.get_barrier_semaphore()
    for nbr in (pos ^ 1, pos ^ 3, pos ^ 4):
        pltpu.semaphore_signal(
            barrier, inc=1, device_id=(nbr,),
            device_id_type=pltpu.DeviceIdType.MESH,
        )
    pltpu.semaphore_wait(barrier, 3)

    offs = [r0 for r0, _ in _PARTS]
    sizes = [n for _, n in _PARTS]
    descs = {}
    addinfo = {}

    def start(p, s):
        order = _ORDERS[p % 3]
        k = p * 6 + s
        if p in _A2A_PARTS:
            r0, n = _PARTS[p]
            m = n // N_DEV
            own = r0 + pos * m
            rd = []
            for d in range(1, N_DEV):
                ks = 36 + (p - 6) * 14 + s * 7 + (d - 1)
                if s == 0:
                    src = c_ref.at[pl.ds(r0 + (pos ^ d) * m, m), :]
                    dst = rbuf.at[pl.ds(_ROFF[(p, 0)] + pos * m, m), :]
                else:
                    src = c_ref.at[pl.ds(own, m), :]
                    dst = c_ref.at[pl.ds(own, m), :]
                rdma = pltpu.make_async_remote_copy(
                    src_ref=src, dst_ref=dst,
                    send_sem=send_sems.at[ks],
                    recv_sem=recv_sems.at[ks],
                    device_id=(pos ^ d,),
                    device_id_type=pltpu.DeviceIdType.MESH,
                )
                rdma.start()
                rd.append(rdma)
            descs[(p, s)] = rd
        elif s < 3:
            half = sizes[p] // 2
            bit = bits[order[s]]
            send_off = offs[p] + (1 - bit) * half
            ro = _ROFF[(p, s)]
            rdma = pltpu.make_async_remote_copy(
                src_ref=c_ref.at[pl.ds(send_off, half), :],
                dst_ref=rbuf.at[pl.ds(ro, half), :],
                send_sem=send_sems.at[k],
                recv_sem=recv_sems.at[k],
                device_id=(pos ^ _XORS[order[s]],),
                device_id_type=pltpu.DeviceIdType.MESH,
            )
            rdma.start()
            offs[p] = offs[p] + bit * half
            sizes[p] = half
            descs[(p, s)] = rdma
            addinfo[(p, s)] = (offs[p], half, ro)
        else:
            ax = order[2 - (s - 3)]
            sz = sizes[p]
            rdma = pltpu.make_async_remote_copy(
                src_ref=c_ref.at[pl.ds(offs[p], sz), :],
                dst_ref=c_ref.at[pl.ds(offs[p], sz), :],
                send_sem=send_sems.at[k],
                recv_sem=recv_sems.at[k],
                device_id=(pos ^ _XORS[ax],),
                device_id_type=pltpu.DeviceIdType.MESH,
            )
            rdma.start()
            offs[p] = offs[p] - bits[ax] * sz
            sizes[p] = sz * 2
            descs[(p, s)] = rdma

    def finish(p, s):
        r0, n = _PARTS[p]
        if p in _A2A_PARTS:
            rd = descs.pop((p, s))
            m = n // N_DEV
            if s == 0:
                for r in rd:
                    r.wait_recv()
                own = r0 + pos * m
                acc = c_ref[pl.ds(own, m), :]
                for d in range(1, N_DEV):
                    ro = _ROFF[(p, 0)]
                    acc = acc + rbuf[pl.ds(ro + (pos ^ d) * m, m), :]
                c_ref[pl.ds(own, m), :] = acc
                descs[(p, "a_sends")] = rd
            else:
                for r in rd:
                    r.wait_recv()
                for r in rd:
                    r.wait_send()
                for r in descs.pop((p, "a_sends")):
                    r.wait_send()
            return
        descs.pop((p, s)).wait()
        if s < 3:
            off, half, ro = addinfo.pop((p, s))
            c_ref[pl.ds(off, half), :] = (
                c_ref[pl.ds(off, half), :] + rbuf[ro:ro + half, :])

    n_slots = max(p + (2 if p in _A2A_PARTS else 6) + 1
                  for p in range(N_PARTS))
    for t in range(n_slots):
        if t < N_PARTS:
            _compute_part(t, x_ref, wq_ref, k_ref, v_ref, wo_ref, c_ref)
        for s in range(7):
            p = t - s
            if p < 0 or p >= N_PARTS:
                continue
            nsteps = 2 if p in _A2A_PARTS else 6
            if 1 <= s <= nsteps:
                finish(p, s - 1)
            if s < nsteps:
                start(p, s)


def kernel(x, Wq, K_ext, V_ext, Wo):
    idx = lax.axis_index("i")
    K_loc = lax.dynamic_slice_in_dim(
        K_ext[0], idx * H_LOC, H_LOC, axis=1
    ).transpose(1, 0, 2).astype(jnp.bfloat16)
    V_loc = lax.dynamic_slice_in_dim(
        V_ext[0], idx * H_LOC, H_LOC, axis=1
    ).transpose(1, 0, 2).astype(jnp.bfloat16)
    out = pl.pallas_call(
        _fused_body,
        out_shape=jax.ShapeDtypeStruct((SQ, D_MODEL), jnp.bfloat16),
        in_specs=[pl.BlockSpec(memory_space=pltpu.VMEM)] * 5,
        out_specs=pl.BlockSpec(memory_space=pltpu.VMEM),
        scratch_shapes=[
            pltpu.VMEM((_RBUF_ROWS, D_MODEL), jnp.bfloat16),
            pltpu.SemaphoreType.DMA((_N_SEMS,)),
            pltpu.SemaphoreType.DMA((_N_SEMS,)),
        ],
        compiler_params=pltpu.CompilerParams(collective_id=0),
    )(x[0], Wq.astype(jnp.bfloat16), K_loc, V_loc, Wo.astype(jnp.bfloat16))
    return out[None]


# device time: 100351 ns/iter; 1.0265x vs baseline; 1.0265x over previous
import jax
import jax.numpy as jnp
from jax import lax
from jax.experimental import pallas as pl
from jax.experimental.pallas import tpu as pltpu

N_DEV = 8
SQ = 2048
SKV = 2048
D_MODEL = 1024
H_LOC = 8
DH = 128
SCALE = 0.08838834764831843
BAND = 128
NGLOB = 32
GLOB = 128

_PARTS = []
_r = 0
for _n in (256, 256, 256, 256, 256, 256, 256, 256):
    _PARTS.append((_r, _n))
    _r += _n
assert _r == SQ
N_PARTS = len(_PARTS)
_A2A_PARTS = frozenset({6, 7})

_ORDERS = (("x", "y", "z"), ("y", "z", "x"), ("z", "x", "y"))
_XORS = {"x": 1, "y": 3, "z": 4}

_ROFF = {}
_acc = 0
for _p, (_r0, _n) in enumerate(_PARTS):
    if _p in _A2A_PARTS:
        _ROFF[(_p, 0)] = _acc
        _acc += _n
    else:
        for _s in range(3):
            _ROFF[(_p, _s)] = _acc
            _acc += _n >> (_s + 1)
_RBUF_ROWS = _acc

_N_SEMS = 36 + 2 * 14


def _softmax_ctx(s, mask, w_slices, v_slices):
    s = jnp.where(mask, s, -1e9)
    w = jnp.exp(s - jnp.max(s, axis=1, keepdims=True))
    w = (w / jnp.sum(w, axis=1, keepdims=True)).astype(jnp.bfloat16)
    ctx = None
    for (lo, hi), v in zip(w_slices, v_slices):
        part = jnp.dot(w[:, lo:hi], v, preferred_element_type=jnp.float32)
        ctx = part if ctx is None else ctx + part
    return ctx.astype(jnp.bfloat16)


def _dot_t(a, bmat):
    return lax.dot_general(
        a, bmat, (((1,), (1,)), ((), ())), preferred_element_type=jnp.float32)


def _compute_part(p, x_ref, wq_ref, k_ref, v_ref, wo_ref, c_ref):
    r0, n = _PARTS[p]
    q = jnp.dot(x_ref[r0:r0 + n, :].astype(jnp.bfloat16), wq_ref[...],
                preferred_element_type=jnp.float32).astype(jnp.bfloat16)
    if p == 0:
        qi = lax.broadcasted_iota(jnp.int32, (n, SKV), 0)
        ki = lax.broadcasted_iota(jnp.int32, (n, SKV), 1)
        mask = (jnp.abs(qi - ki) <= BAND) | (ki < NGLOB) | (qi < NGLOB)
        ctx_parts = []
        for h in range(H_LOC):
            s = _dot_t(q[:, h * DH:(h + 1) * DH], k_ref[h]) * SCALE
            ctx_parts.append(_softmax_ctx(s, mask, [(0, SKV)], [v_ref[h]]))
    else:
        win = n + 2 * BAND
        ws = min(r0 - BAND, SKV - win)
        qi = lax.broadcasted_iota(jnp.int32, (n, win + GLOB), 0) + r0
        ki = lax.broadcasted_iota(jnp.int32, (n, win + GLOB), 1)
        ki = jnp.where(ki < win, ki + ws, ki - win)
        mask = (jnp.abs(qi - ki) <= BAND) | (ki < NGLOB)
        ctx_parts = []
        for h in range(H_LOC):
            qh = q[:, h * DH:(h + 1) * DH]
            s = jnp.concatenate([
                _dot_t(qh, k_ref[h, ws:ws + win, :]),
                _dot_t(qh, k_ref[h, :GLOB, :]),
            ], axis=1) * SCALE
            ctx_parts.append(_softmax_ctx(
                s, mask, [(0, win), (win, win + GLOB)],
                [v_ref[h, ws:ws + win, :], v_ref[h, :GLOB, :]]))
    c_ref[r0:r0 + n, :] = jnp.dot(
        jnp.concatenate(ctx_parts, axis=1), wo_ref[...],
        preferred_element_type=jnp.float32).astype(jnp.bfloat16)


def _fused_body(x_ref, wq_ref, k_ref, v_ref, wo_ref, c_ref,
                rbuf, send_sems, recv_sems):
    pos = lax.axis_index("i")
    cy = (pos >> 1) & 1
    cx = (pos & 1) ^ cy
    cz = (pos >> 2) & 1
    bits = {"x": cx, "y": cy, "z": cz}

    barrier = pltpu.get_barrier_semaphore()
    for nbr in (pos ^ 1, pos ^ 3, pos ^ 4):
        pltpu.semaphore_signal(
            barrier, inc=1, device_id=(nbr,),
            device_id_type=pltpu.DeviceIdType.MESH,
        )
    pltpu.semaphore_wait(barrier, 3)

    offs = [r0 for r0, _ in _PARTS]
    sizes = [n for _, n in _PARTS]
    descs = {}
    addinfo = {}

    def start(p, s):
        order = _ORDERS[p % 3]
        k = p * 6 + s
        if p in _A2A_PARTS:
            r0, n = _PARTS[p]
            m = n // N_DEV
            own = r0 + pos * m
            rd = []
            for d in range(1, N_DEV):
                ks = 36 + (p - 6) * 14 + s * 7 + (d - 1)
                if s == 0:
                    src = c_ref.at[pl.ds(r0 + (pos ^ d) * m, m), :]
                    dst = rbuf.at[pl.ds(_ROFF[(p, 0)] + pos * m, m), :]
                else:
                    src = c_ref.at[pl.ds(own, m), :]
                    dst = c_ref.at[pl.ds(own, m), :]
                rdma = pltpu.make_async_remote_copy(
                    src_ref=src, dst_ref=dst,
                    send_sem=send_sems.at[ks],
                    recv_sem=recv_sems.at[ks],
                    device_id=(pos ^ d,),
                    device_id_type=pltpu.DeviceIdType.MESH,
                )
                rdma.start()
                rd.append(rdma)
            descs[(p, s)] = rd
        elif s < 3:
            half = sizes[p] // 2
            bit = bits[order[s]]
            send_off = offs[p] + (1 - bit) * half
            ro = _ROFF[(p, s)]
            rdma = pltpu.make_async_remote_copy(
                src_ref=c_ref.at[pl.ds(send_off, half), :],
                dst_ref=rbuf.at[pl.ds(ro, half), :],
                send_sem=send_sems.at[k],
                recv_sem=recv_sems.at[k],
                device_id=(pos ^ _XORS[order[s]],),
                device_id_type=pltpu.DeviceIdType.MESH,
            )
            rdma.start()
            offs[p] = offs[p] + bit * half
            sizes[p] = half
            descs[(p, s)] = rdma
            addinfo[(p, s)] = (offs[p], half, ro)
        else:
            ax = order[2 - (s - 3)]
            sz = sizes[p]
            rdma = pltpu.make_async_remote_copy(
                src_ref=c_ref.at[pl.ds(offs[p], sz), :],
                dst_ref=c_ref.at[pl.ds(offs[p], sz), :],
                send_sem=send_sems.at[k],
                recv_sem=recv_sems.at[k],
                device_id=(pos ^ _XORS[ax],),
                device_id_type=pltpu.DeviceIdType.MESH,
            )
            rdma.start()
            offs[p] = offs[p] - bits[ax] * sz
            sizes[p] = sz * 2
            descs[(p, s)] = rdma

    def finish(p, s):
        r0, n = _PARTS[p]
        if p in _A2A_PARTS:
            rd = descs.pop((p, s))
            m = n // N_DEV
            if s == 0:
                for r in rd:
                    r.wait_recv()
                own = r0 + pos * m
                acc = c_ref[pl.ds(own, m), :]
                for d in range(1, N_DEV):
                    ro = _ROFF[(p, 0)]
                    acc = acc + rbuf[pl.ds(ro + (pos ^ d) * m, m), :]
                c_ref[pl.ds(own, m), :] = acc
                descs[(p, "a_sends")] = rd
            else:
                for r in rd:
                    r.wait_recv()
                for r in rd:
                    r.wait_send()
                for r in descs.pop((p, "a_sends")):
                    r.wait_send()
            return
        descs.pop((p, s)).wait()
        if s < 3:
            off, half, ro = addinfo.pop((p, s))
            c_ref[pl.ds(off, half), :] = (
                c_ref[pl.ds(off, half), :] + rbuf[ro:ro + half, :])

    n_slots = max(p + (2 if p in _A2A_PARTS else 6) + 1
                  for p in range(N_PARTS))
    for t in range(n_slots):
        if t < N_PARTS:
            _compute_part(t, x_ref, wq_ref, k_ref, v_ref, wo_ref, c_ref)
        for s in range(7):
            p = t - s
            if p < 0 or p >= N_PARTS:
                continue
            nsteps = 2 if p in _A2A_PARTS else 6
            if 1 <= s <= nsteps:
                finish(p, s - 1)
            if s < nsteps:
                start(p, s)


def kernel(x, Wq, K_ext, V_ext, Wo):
    idx = lax.axis_index("i")
    K_loc = lax.dynamic_slice_in_dim(
        K_ext[0], idx * H_LOC, H_LOC, axis=1
    ).transpose(1, 0, 2).astype(jnp.bfloat16)
    V_loc = lax.dynamic_slice_in_dim(
        V_ext[0], idx * H_LOC, H_LOC, axis=1
    ).transpose(1, 0, 2).astype(jnp.bfloat16)
    out = pl.pallas_call(
        _fused_body,
        out_shape=jax.ShapeDtypeStruct((SQ, D_MODEL), jnp.bfloat16),
        in_specs=[pl.BlockSpec(memory_space=pltpu.VMEM)] * 5,
        out_specs=pl.BlockSpec(memory_space=pltpu.VMEM),
        scratch_shapes=[
            pltpu.VMEM((_RBUF_ROWS, D_MODEL), jnp.bfloat16),
            pltpu.SemaphoreType.DMA((_N_SEMS,)),
            pltpu.SemaphoreType.DMA((_N_SEMS,)),
        ],
        compiler_params=pltpu.CompilerParams(collective_id=0),
    )(x[0], Wq.astype(jnp.bfloat16), K_loc, V_loc, Wo.astype(jnp.bfloat16))
    return out[None]
